# baseline (speedup 1.0000x reference)
import functools

import jax
import jax.numpy as jnp
from jax import lax
from jax.experimental import pallas as pl
from jax.experimental.pallas import tpu as pltpu
from jax.experimental.pallas import tpu_sc as plsc

EMB = 32
B = 16384

_info = plsc.get_sparse_core_info()
_NC, _NS, _L = _info.num_cores, _info.num_subcores, _info.num_lanes
_NW = _NC * _NS
_BW = B // _NW

_mesh = plsc.VectorSubcoreMesh(core_axis_name="c", subcore_axis_name="s")

_f32 = jnp.float32
_i32 = jnp.int32


@functools.partial(
    pl.kernel,
    mesh=_mesh,
    compiler_params=pltpu.CompilerParams(use_tc_tiling_on_sc=False),
    out_type=[
        jax.ShapeDtypeStruct((B, EMB), _f32),
        jax.ShapeDtypeStruct((B, EMB), _f32),
        jax.ShapeDtypeStruct((B, EMB), _f32),
        jax.ShapeDtypeStruct((B, EMB), _f32),
    ],
    scratch_types=[
        pltpu.VMEM((_BW,), _i32),
        pltpu.VMEM((_BW,), _i32),
        pltpu.VMEM((_BW,), _i32),
        pltpu.VMEM((_BW,), _i32),
        pltpu.VMEM((_BW, EMB), _f32),
        pltpu.VMEM((_BW, EMB), _f32),
        pltpu.VMEM((_BW, EMB), _f32),
        pltpu.VMEM((_BW, EMB), _f32),
        pltpu.SemaphoreType.DMA,
        pltpu.SemaphoreType.DMA,
    ],
)
def _sc_gather(idx_us_h, idx_is_h, idx_ut_h, idx_it_h,
               tab_us_h, tab_is_h, tab_ut_h, tab_it_h,
               out_us_h, out_is_h, out_ut_h, out_it_h,
               idx_us, idx_is, idx_ut, idx_it,
               rows_us, rows_is, rows_ut, rows_it,
               gsem, wsem):
    wid = lax.axis_index("s") * _NC + lax.axis_index("c")
    base = wid * _BW

    pltpu.sync_copy(idx_us_h.at[pl.ds(base, _BW)], idx_us)
    pltpu.sync_copy(idx_is_h.at[pl.ds(base, _BW)], idx_is)
    pltpu.sync_copy(idx_ut_h.at[pl.ds(base, _BW)], idx_ut)
    pltpu.sync_copy(idx_it_h.at[pl.ds(base, _BW)], idx_it)

    c0 = pltpu.async_copy(tab_us_h.at[idx_us], rows_us, gsem)
    c1 = pltpu.async_copy(tab_is_h.at[idx_is], rows_is, gsem)
    c2 = pltpu.async_copy(tab_ut_h.at[idx_ut], rows_ut, gsem)
    c3 = pltpu.async_copy(tab_it_h.at[idx_it], rows_it, gsem)

    c0.wait()
    w0 = pltpu.async_copy(rows_us, out_us_h.at[pl.ds(base, _BW)], wsem)
    c1.wait()
    w1 = pltpu.async_copy(rows_is, out_is_h.at[pl.ds(base, _BW)], wsem)
    c2.wait()
    w2 = pltpu.async_copy(rows_ut, out_ut_h.at[pl.ds(base, _BW)], wsem)
    c3.wait()
    w3 = pltpu.async_copy(rows_it, out_it_h.at[pl.ds(base, _BW)], wsem)

    w0.wait()
    w1.wait()
    w2.wait()
    w3.wait()


def _score_body(us_ref, is_ref, ut_ref, it_ref, ss_ref, st_ref):
    ss_ref[...] = jnp.sum(us_ref[...] * is_ref[...], axis=1)
    st_ref[...] = jnp.sum(ut_ref[...] * it_ref[...], axis=1)


_tc_scores = pl.pallas_call(
    _score_body,
    out_shape=[
        jax.ShapeDtypeStruct((B,), _f32),
        jax.ShapeDtypeStruct((B,), _f32),
    ],
)


def kernel(mirnas_s, disease, label_s, mirnas_t, target, label_t,
           mirna_emb_s, item_emb_s, mirna_emb_t, item_emb_t):
    idx_us = mirnas_s.astype(_i32)
    idx_is = disease.astype(_i32)
    idx_ut = mirnas_t.astype(_i32)
    idx_it = target.astype(_i32)
    u_s, i_s, u_t, i_t = _sc_gather(
        idx_us, idx_is, idx_ut, idx_it,
        mirna_emb_s, item_emb_s, mirna_emb_t, item_emb_t)
    scores_s, scores_t = _tc_scores(u_s, i_s, u_t, i_t)
    return (scores_s, scores_t,
            (u_s, i_s, label_s),
            (u_t, i_t, label_t))
